# baseline (speedup 1.0000x reference)
import jax
import jax.numpy as jnp
from jax import lax
from jax.experimental import pallas as pl
from jax.experimental.pallas import tpu as pltpu
from jax.experimental.pallas import tpu_sc as plsc

L = 16
C = 32768
NV = C // L
G = 16
ROWS = 1024
NW = 32
KTOP = 8
NEG = float("-inf")
NCH = 4
CHUNK = ROWS // NCH
RPWC = CHUNK // NW
SUB = 256
LANE = 128


def _tree_max(vs):
    vs = list(vs)
    while len(vs) > 1:
        nxt = [jnp.maximum(vs[i], vs[i + 1]) for i in range(0, len(vs) - 1, 2)]
        if len(vs) % 2:
            nxt.append(vs[-1])
        vs = nxt
    return vs[0]


def _sc_thr_body(chunk_base):
    def body(in_hbm, thr_hbm, bufA, bufB, thr_v, lsemA, lsemB):
        wid = lax.axis_index("s") * 2 + lax.axis_index("c")
        base = chunk_base + wid * RPWC
        last = base + RPWC - 1

        pltpu.async_copy(in_hbm.at[base], bufA, lsemA)
        pltpu.async_copy(in_hbm.at[base + 1], bufB, lsemB)

        def process(i, buf, lsem):
            row = base + i
            pltpu.make_async_copy(in_hbm.at[row], buf, lsem).wait()

            init = tuple(jnp.full((L,), NEG) for _ in range(2 * G))

            def p1(j, st):
                cs = list(st[:G])
                ds = list(st[G:])
                for g in range(G):
                    v = buf[pl.ds(j * G * L + g * L, L)]
                    lo = jnp.minimum(cs[g], v)
                    cs[g] = jnp.maximum(cs[g], v)
                    ds[g] = jnp.maximum(ds[g], lo)
                return tuple(cs) + tuple(ds)

            pool = lax.fori_loop(0, NV // G, p1, init)

            t = jnp.float32(float("inf"))
            for _ in range(KTOP):
                masked = [jnp.where(p < t, p, NEG) for p in pool]
                t = jnp.max(_tree_max(masked))
            thr_v[pl.ds(i * L, L)] = jnp.full((L,), t)

            nxt = jnp.minimum(row + 2, last)
            pltpu.async_copy(in_hbm.at[nxt], buf, lsem)

        def pair(i, carry):
            process(2 * i, bufA, lsemA)
            process(2 * i + 1, bufB, lsemB)
            return carry

        lax.fori_loop(0, RPWC // 2, pair, jnp.int32(0))

        pltpu.make_async_copy(in_hbm.at[last], bufA, lsemA).wait()
        pltpu.make_async_copy(in_hbm.at[last], bufB, lsemB).wait()
        pltpu.sync_copy(thr_v, thr_hbm.at[pl.ds(wid * RPWC * L, RPWC * L)])

    return body


def _sc_thresholds(flat, chunk_base):
    mesh = plsc.VectorSubcoreMesh(core_axis_name="c", subcore_axis_name="s")
    return pl.kernel(
        _sc_thr_body(chunk_base),
        out_type=jax.ShapeDtypeStruct((CHUNK * L,), jnp.float32),
        mesh=mesh,
        compiler_params=pltpu.CompilerParams(needs_layout_passes=False),
        scratch_types=[
            pltpu.VMEM((C,), jnp.float32),
            pltpu.VMEM((C,), jnp.float32),
            pltpu.VMEM((RPWC * L,), jnp.float32),
            pltpu.SemaphoreType.DMA,
            pltpu.SemaphoreType.DMA,
        ],
    )(flat)


def _tc_mask_body(has_prev):
    def body(*refs):
        x_ref, thr_ref, kv_ref = refs[0], refs[1], refs[2]
        o_ref = refs[-1]
        t = thr_ref[0, 0, 0]
        kvs = kv_ref[0, 0, 0]
        x = x_ref[0]
        m = x >= t
        count = jnp.sum(m.astype(jnp.int32))

        @pl.when(count == KTOP)
        def _common():
            o_ref[0] = jnp.where(m, x, 0.0) + kvs

        @pl.when(count != KTOP)
        def _fallback():
            def cond(st):
                return st[1] < KTOP

            def wbody(st):
                tc, cgt, _tp, _cp = st
                mv = jnp.max(jnp.where(x < tc, x, NEG))
                ce = jnp.sum((x == tc).astype(jnp.int32))
                return (mv, cgt + ce, tc, cgt)

            tinf = jnp.float32(float("inf"))
            st = lax.while_loop(
                cond, wbody, (tinf, jnp.int32(0), tinf, jnp.int32(0)))
            t8x = st[2]
            need = KTOP - st[3]
            eq = x == t8x
            flat_i = (lax.broadcasted_iota(jnp.int32, (SUB, LANE), 0) * LANE
                      + lax.broadcasted_iota(jnp.int32, (SUB, LANE), 1))
            big = jnp.int32(C)

            def mstep(j, cutoff):
                m2 = jnp.logical_and(eq, flat_i > cutoff)
                nxt = jnp.min(jnp.where(m2, flat_i, big))
                return jnp.where(j < need, nxt, cutoff)

            cutoff = lax.fori_loop(0, KTOP, mstep, jnp.int32(-1))
            keep = jnp.logical_or(
                x > t8x, jnp.logical_and(eq, flat_i <= cutoff))
            o_ref[0] = jnp.where(keep, x, 0.0) + kvs

    return body


def _tc_mask(x3d, thr, kv2d, chunk_base, prev):
    thr2d = thr.reshape(CHUNK, 1, L)
    in_specs = [
        pl.BlockSpec((1, SUB, LANE), lambda i: (chunk_base + i, 0, 0)),
        pl.BlockSpec((1, 1, L), lambda i: (i, 0, 0)),
        pl.BlockSpec((1, 1, L), lambda i: (0, 0, 0)),
    ]
    args = [x3d, thr2d, kv2d]
    aliases = {}
    if prev is not None:
        in_specs.append(pl.BlockSpec(memory_space=pl.ANY))
        args.append(prev)
        aliases = {3: 0}
    return pl.pallas_call(
        _tc_mask_body(prev is not None),
        grid=(CHUNK,),
        in_specs=in_specs,
        out_specs=pl.BlockSpec((1, SUB, LANE), lambda i: (chunk_base + i, 0, 0)),
        out_shape=jax.ShapeDtypeStruct((ROWS, SUB, LANE), jnp.float32),
        input_output_aliases=aliases,
    )(*args)


def kernel(t, K):
    B, R, Cc = t.shape
    flat = t.reshape(B * R, Cc)
    x3d = flat.reshape(ROWS, SUB, LANE)
    kv2d = jnp.full((1, 1, L), jnp.asarray(K, jnp.float32) - float(KTOP),
                    dtype=jnp.float32)

    thrs = [_sc_thresholds(flat, c * CHUNK) for c in range(NCH)]
    out = None
    for c in range(NCH):
        out = _tc_mask(x3d, thrs[c], kv2d, c * CHUNK, out)
    return out.reshape(B, R, Cc)
